# baseline (speedup 1.0000x reference)
import functools

import jax
import jax.numpy as jnp
from jax import lax
from jax.experimental import pallas as pl
from jax.experimental.pallas import tpu as pltpu
from jax.experimental.pallas import tpu_sc as plsc

B, S, EMB = 16, 1024, 1024
NC, NS = 2, 16
NW = NC * NS
SPW = S // NW
CH = 16
NCHS = SPW // CH
NP = B
NG = NP // NCHS
LANES = 16
KPC = EMB // LANES

_mesh = plsc.VectorSubcoreMesh(core_axis_name="c", subcore_axis_name="s")


@functools.partial(
    pl.kernel,
    out_type=jax.ShapeDtypeStruct((B * S, EMB), jnp.float32),
    mesh=_mesh,
    scratch_types=[
        pltpu.VMEM((NP, 2 * CH), jnp.int32),
        pltpu.VMEM((SPW, EMB), jnp.float32),
        pltpu.VMEM((2, 2 * CH, EMB), jnp.float32),
        [pltpu.SemaphoreType.DMA] * 2,
        [pltpu.SemaphoreType.DMA] * 2,
    ],
)
def _emb_lookup(ids_hbm, tok_hbm, pos_hbm, out_hbm, idx_v, pos_v, buf_v,
                gat_sems, out_sems):
    wid = lax.axis_index("s") * NC + lax.axis_index("c")
    s_base = wid * SPW

    pltpu.sync_copy(ids_hbm.at[wid], idx_v)

    def start_gather(p, ss):
        return pltpu.async_copy(
            tok_hbm.at[idx_v.at[p]], buf_v.at[ss], gat_sems[ss])

    def wait_gather(p, ss):
        pltpu.make_async_copy(
            tok_hbm.at[idx_v.at[p]], buf_v.at[ss], gat_sems[ss]).wait()

    def out_rows(p, e, u):
        return (p - e + u) * S + s_base + e * CH

    def start_outs(p, e, ss):
        for u in range(2):
            pltpu.async_copy(
                buf_v.at[ss, pl.ds(u * CH, CH)],
                out_hbm.at[pl.ds(out_rows(p, e, u), CH)], out_sems[ss])

    def wait_outs(p, e, ss):
        for u in range(2):
            pltpu.make_async_copy(
                buf_v.at[ss, pl.ds(u * CH, CH)],
                out_hbm.at[pl.ds(out_rows(p, e, u), CH)], out_sems[ss]).wait()

    def add_pair(e, ss):
        @plsc.parallel_loop(0, CH, unroll=2)
        def _(r):
            prow = e * CH + r
            for k in range(KPC):
                sl = pl.ds(k * LANES, LANES)
                pvec = pos_v[prow, sl]
                plsc.addupdate(buf_v.at[ss, r, sl], pvec)
                plsc.addupdate(buf_v.at[ss, CH + r, sl], pvec)

    start_gather(0, 0)
    pltpu.sync_copy(pos_hbm.at[pl.ds(s_base, SPW)], pos_v)

    @pl.loop(0, NG)
    def _(g):
        wait_gather(2 * g, 0)
        @pl.when(g > 0)
        def _():
            wait_outs(2 * g - 1, 1, 1)
        start_gather(2 * g + 1, 1)
        add_pair(0, 0)
        start_outs(2 * g, 0, 0)

        wait_gather(2 * g + 1, 1)
        @pl.when(g < NG - 1)
        def _():
            wait_outs(2 * g, 0, 0)
            start_gather(2 * g + 2, 0)
        add_pair(1, 1)
        start_outs(2 * g + 1, 1, 1)

    wait_outs(NP - 2, 0, 0)
    wait_outs(NP - 1, 1, 1)


def kernel(input_ids, tok_table, pos_table):
    ids4 = (input_ids.astype(jnp.int32)
            .reshape(NG, 2, NW, NCHS, CH)
            .transpose(2, 0, 3, 1, 4)
            .reshape(NW, NP, 2 * CH))
    out = _emb_lookup(ids4, tok_table, pos_table)
    return out.reshape(B, S, EMB)


# speedup vs baseline: 1.0448x; 1.0448x over previous
import functools

import jax
import jax.numpy as jnp
from jax import lax
from jax.experimental import pallas as pl
from jax.experimental.pallas import tpu as pltpu
from jax.experimental.pallas import tpu_sc as plsc

B, S, EMB = 16, 1024, 1024
NC, NS = 2, 16
NW = NC * NS
SPW = S // NW
CH = 16
NCHS = SPW // CH
NCHUNK = B * NCHS
NBUF = 4
NG = NCHUNK // NBUF
LANES = 16
KPC = EMB // LANES

_mesh = plsc.VectorSubcoreMesh(core_axis_name="c", subcore_axis_name="s")


@functools.partial(
    pl.kernel,
    out_type=jax.ShapeDtypeStruct((B * S, EMB), jnp.float32),
    mesh=_mesh,
    scratch_types=[
        pltpu.VMEM((NCHUNK, CH), jnp.int32),
        pltpu.VMEM((SPW, EMB), jnp.float32),
        pltpu.VMEM((NBUF, CH, EMB), jnp.float32),
        [pltpu.SemaphoreType.DMA] * NBUF,
        [pltpu.SemaphoreType.DMA] * NBUF,
    ],
)
def _emb_lookup(ids_hbm, tok_hbm, pos_hbm, out_hbm, idx_v, pos_v, buf_v,
                gat_sems, out_sems):
    wid = lax.axis_index("s") * NC + lax.axis_index("c")
    s_base = wid * SPW

    pltpu.sync_copy(ids_hbm.at[wid], idx_v)

    def start_gather(c, slot):
        return pltpu.async_copy(
            tok_hbm.at[idx_v.at[c]], buf_v.at[slot], gat_sems[slot])

    def out_rows(c, h):
        return (c - h) * (S // NCHS) + s_base + h * CH

    def start_out(c, h, slot):
        return pltpu.async_copy(
            buf_v.at[slot],
            out_hbm.at[pl.ds(out_rows(c, h), CH)], out_sems[slot])

    def wait_gather(c, slot):
        pltpu.make_async_copy(
            tok_hbm.at[pl.ds(0, CH)], buf_v.at[slot], gat_sems[slot]).wait()

    def wait_out(c, h, slot):
        pltpu.make_async_copy(
            buf_v.at[slot],
            out_hbm.at[pl.ds(0, CH)], out_sems[slot]).wait()

    def add_pair(h, s0, s1):
        @plsc.parallel_loop(0, CH, unroll=2)
        def _(r):
            prow = h * CH + r
            for k in range(KPC):
                sl = pl.ds(k * LANES, LANES)
                pvec = pos_v[prow, sl]
                plsc.addupdate(buf_v.at[s0, r, sl], pvec)
                plsc.addupdate(buf_v.at[s1, r, sl], pvec)

    start_gather(0, 0)
    start_gather(2, 2)
    pltpu.sync_copy(pos_hbm.at[pl.ds(s_base, SPW)], pos_v)

    @pl.loop(0, NG)
    def _(g):
        wait_gather(4 * g, 0)
        wait_gather(4 * g + 2, 2)
        @pl.when(g > 0)
        def _():
            wait_out(4 * g - 3, 1, 1)
            wait_out(4 * g - 1, 1, 3)
        start_gather(4 * g + 1, 1)
        start_gather(4 * g + 3, 3)
        add_pair(0, 0, 2)
        start_out(4 * g, 0, 0)
        start_out(4 * g + 2, 0, 2)

        wait_gather(4 * g + 1, 1)
        wait_gather(4 * g + 3, 3)
        @pl.when(g < NG - 1)
        def _():
            wait_out(4 * g, 0, 0)
            wait_out(4 * g + 2, 0, 2)
            start_gather(4 * g + 4, 0)
            start_gather(4 * g + 6, 2)
        add_pair(1, 1, 3)
        start_out(4 * g + 1, 1, 1)
        start_out(4 * g + 3, 1, 3)

    wait_out(NCHUNK - 4, 0, 0)
    wait_out(NCHUNK - 2, 0, 2)
    wait_out(NCHUNK - 3, 1, 1)
    wait_out(NCHUNK - 1, 1, 3)


def kernel(input_ids, tok_table, pos_table):
    ids3 = (input_ids.astype(jnp.int32)
            .reshape(B, NW, NCHS, CH)
            .transpose(1, 0, 2, 3)
            .reshape(NW, NCHUNK, CH))
    out = _emb_lookup(ids3, tok_table, pos_table)
    return out.reshape(B, S, EMB)


# speedup vs baseline: 1.1294x; 1.0810x over previous
import functools

import jax
import jax.numpy as jnp
from jax import lax
from jax.experimental import pallas as pl
from jax.experimental.pallas import tpu as pltpu
from jax.experimental.pallas import tpu_sc as plsc

B, S, EMB = 16, 1024, 1024
NC, NS = 2, 16
NW = NC * NS
SPW = S // NW
CH = 16
NCHS = SPW // CH
NCHUNK = B * NCHS
NBUF = 4
NG = NCHUNK // NBUF
LANES = 16
KPC = EMB // LANES

_mesh = plsc.VectorSubcoreMesh(core_axis_name="c", subcore_axis_name="s")


@functools.partial(
    pl.kernel,
    out_type=jax.ShapeDtypeStruct((B * S, EMB), jnp.float32),
    mesh=_mesh,
    scratch_types=[
        pltpu.VMEM((NCHUNK, CH), jnp.int32),
        pltpu.VMEM((SPW, EMB), jnp.float32),
        pltpu.VMEM((NBUF, CH, EMB), jnp.float32),
        [pltpu.SemaphoreType.DMA] * NBUF,
        [pltpu.SemaphoreType.DMA] * NBUF,
    ],
)
def _emb_lookup(ids_hbm, tok_hbm, pos_hbm, out_hbm, idx_v, pos_v, buf_v,
                gat_sems, out_sems):
    wid = lax.axis_index("s") * NC + lax.axis_index("c")
    s_base = wid * SPW

    pltpu.sync_copy(ids_hbm.at[wid], idx_v)

    def start_gather(c, slot):
        return pltpu.async_copy(
            tok_hbm.at[idx_v.at[c]], buf_v.at[slot], gat_sems[slot])

    def out_rows(c, h):
        return (c - h) * (S // NCHS) + s_base + h * CH

    def start_out(c, h, slot):
        return pltpu.async_copy(
            buf_v.at[slot],
            out_hbm.at[pl.ds(out_rows(c, h), CH)], out_sems[slot])

    def wait_gather(c, slot):
        pltpu.make_async_copy(
            tok_hbm.at[pl.ds(0, CH)], buf_v.at[slot], gat_sems[slot]).wait()

    def wait_out(c, h, slot):
        pltpu.make_async_copy(
            buf_v.at[slot],
            out_hbm.at[pl.ds(0, CH)], out_sems[slot]).wait()

    def add_pair(h, s0, s1):
        @plsc.parallel_loop(0, CH, unroll=1)
        def _(r):
            prow = h * CH + r
            for k in range(KPC):
                sl = pl.ds(k * LANES, LANES)
                pvec = pos_v[prow, sl]
                plsc.addupdate(buf_v.at[s0, r, sl], pvec)
                plsc.addupdate(buf_v.at[s1, r, sl], pvec)

    start_gather(0, 0)
    start_gather(2, 2)
    pltpu.sync_copy(pos_hbm.at[pl.ds(s_base, SPW)], pos_v)

    @pl.loop(0, NG)
    def _(g):
        wait_gather(4 * g, 0)
        wait_gather(4 * g + 2, 2)
        @pl.when(g > 0)
        def _():
            wait_out(4 * g - 3, 1, 1)
            wait_out(4 * g - 1, 1, 3)
        start_gather(4 * g + 1, 1)
        start_gather(4 * g + 3, 3)
        add_pair(0, 0, 2)
        start_out(4 * g, 0, 0)
        start_out(4 * g + 2, 0, 2)

        wait_gather(4 * g + 1, 1)
        wait_gather(4 * g + 3, 3)
        @pl.when(g < NG - 1)
        def _():
            wait_out(4 * g, 0, 0)
            wait_out(4 * g + 2, 0, 2)
            start_gather(4 * g + 4, 0)
            start_gather(4 * g + 6, 2)
        add_pair(1, 1, 3)
        start_out(4 * g + 1, 1, 1)
        start_out(4 * g + 3, 1, 3)

    wait_out(NCHUNK - 4, 0, 0)
    wait_out(NCHUNK - 2, 0, 2)
    wait_out(NCHUNK - 3, 1, 1)
    wait_out(NCHUNK - 1, 1, 3)


def kernel(input_ids, tok_table, pos_table):
    ids3 = (input_ids.astype(jnp.int32)
            .reshape(B, NW, NCHS, CH)
            .transpose(1, 0, 2, 3)
            .reshape(NW, NCHUNK, CH))
    out = _emb_lookup(ids3, tok_table, pos_table)
    return out.reshape(B, S, EMB)
